# baseline (speedup 1.0000x reference)
import functools

import jax
import jax.numpy as jnp
from jax import lax
from jax.experimental import pallas as pl
from jax.experimental.pallas import tpu as pltpu
from jax.experimental.pallas import tpu_sc as plsc

NC = 2
NS = 16
LN = 16
NW = NC * NS
CH = 128
KF = 4
DS = 32


def _sc_mesh():
    return plsc.VectorSubcoreMesh(
        core_axis_name="c", subcore_axis_name="s",
        num_cores=NC, num_subcores=NS)


def _degree_kernel(col2, n_hist, cpw):
    rpt = n_hist // NS

    @functools.partial(
        pl.kernel,
        out_type=jax.ShapeDtypeStruct((NC, n_hist), jnp.float32),
        mesh=_sc_mesh(),
        compiler_params=pltpu.CompilerParams(use_tc_tiling_on_sc=False),
        scratch_types=[
            pltpu.VMEM((cpw, CH), jnp.int32),
            pltpu.VMEM((CH,), jnp.float32),
            pltpu.VMEM((rpt,), jnp.float32),
            pltpu.VMEM_SHARED((n_hist,), jnp.float32),
        ],
    )
    def k(col_hbm, deg_hbm, colbuf, ones_v, tbuf, deg_sh):
        ci = lax.axis_index("c")
        s = lax.axis_index("s")
        w = s * NC + ci
        for j in range(CH // LN):
            ones_v[pl.ds(j * LN, LN)] = jnp.ones((LN,), jnp.float32)

        def zero_body(i, _):
            tbuf[pl.ds(i * LN, LN)] = jnp.zeros((LN,), jnp.float32)
            return 0
        lax.fori_loop(0, rpt // LN, zero_body, 0)
        pltpu.sync_copy(tbuf, deg_sh.at[pl.ds(s * rpt, rpt)])
        pltpu.sync_copy(col_hbm.at[pl.ds(w * cpw, cpw)], colbuf)
        plsc.subcore_barrier()

        def body(ch, _):
            pltpu.sync_copy(ones_v, deg_sh.at[colbuf.at[ch]], add=True)
            return 0
        lax.fori_loop(0, cpw, body, 0)
        plsc.subcore_barrier()

        pltpu.sync_copy(deg_sh.at[pl.ds(s * rpt, rpt)], tbuf)
        pltpu.sync_copy(tbuf, deg_hbm.at[ci, pl.ds(s * rpt, rpt)])

    return k(col2)


NB = 4


def _edge_kernel(ytabs, row2, col2, n_hist, cpw):
    rpt = n_hist // NS
    ngrp = cpw // NB
    assert ngrp % 2 == 0
    last = cpw - NB

    @functools.partial(
        pl.kernel,
        out_type=[jax.ShapeDtypeStruct((NC, n_hist, DS), jnp.float32)
                  for _ in range(KF)],
        mesh=_sc_mesh(),
        compiler_params=pltpu.CompilerParams(use_tc_tiling_on_sc=False),
        scratch_types=[
            pltpu.VMEM((cpw, CH), jnp.int32),
            pltpu.VMEM((cpw, CH), jnp.int32),
            [[pltpu.VMEM((CH, DS), jnp.float32) for _ in range(NB)]
             for _ in range(2)],
            pltpu.VMEM((rpt, DS), jnp.float32),
            pltpu.VMEM_SHARED((n_hist, DS), jnp.float32),
            [pltpu.SemaphoreType.DMA for _ in range(2)],
        ],
    )
    def k(y0, y1, y2, y3, row_hbm, col_hbm, a0, a1, a2, a3,
          rowbuf, colbuf, bufs, zbuf, acc_sh, sems):
        ci = lax.axis_index("c")
        s = lax.axis_index("s")
        w = s * NC + ci
        ys = [y0, y1, y2, y3]
        accs = [a0, a1, a2, a3]

        pltpu.sync_copy(row_hbm.at[pl.ds(w * cpw, cpw)], rowbuf)
        pltpu.sync_copy(col_hbm.at[pl.ds(w * cpw, cpw)], colbuf)

        def zero_body(i, _):
            for j in range(DS // LN):
                zbuf[i, pl.ds(j * LN, LN)] = jnp.zeros((LN,), jnp.float32)
            return 0
        lax.fori_loop(0, rpt, zero_body, 0)

        for kk in range(KF):
            pltpu.sync_copy(zbuf, acc_sh.at[pl.ds(s * rpt, rpt)])
            plsc.subcore_barrier()
            y_t = ys[kk]

            def body(ch, _, _y=y_t):
                pltpu.async_copy(_y.at[rowbuf.at[ch]], bufs[0][0],
                                 sems[0]).wait()
                pltpu.sync_copy(bufs[0][0], acc_sh.at[colbuf.at[ch]],
                                add=True)
                return 0
            lax.fori_loop(0, cpw, body, 0)
            plsc.subcore_barrier()

            pltpu.sync_copy(acc_sh.at[pl.ds(s * rpt, rpt)], zbuf)
            pltpu.sync_copy(zbuf, accs[kk].at[ci, pl.ds(s * rpt, rpt)])
            if kk + 1 < KF:
                lax.fori_loop(0, rpt, zero_body, 0)

    return k(*ytabs, row2, col2)


def _pre_kernel(x, w_gcn, dega, degb, blk):
    n, d = x.shape
    n_hist = dega.shape[0]
    grid = (n + blk - 1) // blk

    def body(x_ref, w_ref, da_ref, db_ref, y_ref):
        i = pl.program_id(0)
        deg = da_ref[pl.ds(i * blk, blk)] + db_ref[pl.ds(i * blk, blk)] + 1.0
        dinv = lax.rsqrt(deg)
        xw = jnp.dot(x_ref[...], w_ref[...],
                     preferred_element_type=jnp.float32)
        y_ref[...] = xw * dinv[:, None]

    return pl.pallas_call(
        body,
        grid=(grid,),
        in_specs=[
            pl.BlockSpec((blk, d), lambda i: (i, 0)),
            pl.BlockSpec((d, d), lambda i: (0, 0)),
            pl.BlockSpec((n_hist,), lambda i: (0,)),
            pl.BlockSpec((n_hist,), lambda i: (0,)),
        ],
        out_specs=pl.BlockSpec((blk, d), lambda i: (i, 0)),
        out_shape=jax.ShapeDtypeStruct((n, d), jnp.float32),
    )(x, w_gcn, dega, degb)


def _final_kernel(accs, y4, dega, degb, w_proj, bg4, s14, t14, s2f, tb2f,
                  blk):
    n = y4.shape[1]
    d = KF * DS
    n_hist = dega.shape[0]
    grid = (n + blk - 1) // blk

    def body(a0, a1, a2, a3, y_ref, da_ref, db_ref, w_ref,
             bg_ref, s1_ref, t1_ref, s2_ref, tb2_ref, h_ref, p_ref):
        i = pl.program_id(0)
        deg = da_ref[pl.ds(i * blk, blk)] + db_ref[pl.ds(i * blk, blk)] + 1.0
        dinv = lax.rsqrt(deg)[:, None]
        acc_refs = [a0, a1, a2, a3]
        z = jnp.zeros((blk, d), jnp.float32)
        for kk in range(KF):
            acc = acc_refs[kk][0] + acc_refs[kk][1] + y_ref[kk]
            b_gcn = bg_ref[kk, 0, :][None, :]
            s1 = s1_ref[kk, 0, :][None, :]
            t1 = t1_ref[kk, 0, :][None, :]
            h_k = (acc * dinv + b_gcn) * s1 + t1
            h_ref[kk] = h_k
            z = z + jnp.dot(h_k, w_ref[pl.ds(kk * DS, DS), :],
                            preferred_element_type=jnp.float32)
        p_ref[...] = jnp.maximum(z * s2_ref[0, :][None, :]
                                 + tb2_ref[0, :][None, :], 0.0)

    vec4 = pl.BlockSpec((KF, 1, DS), lambda i: (0, 0, 0))
    vecd = pl.BlockSpec((1, d), lambda i: (0, 0))
    return pl.pallas_call(
        body,
        grid=(grid,),
        in_specs=(
            [pl.BlockSpec((NC, blk, DS), lambda i: (0, i, 0))
             for _ in range(KF)]
            + [
                pl.BlockSpec((KF, blk, DS), lambda i: (0, i, 0)),
                pl.BlockSpec((n_hist,), lambda i: (0,)),
                pl.BlockSpec((n_hist,), lambda i: (0,)),
                pl.BlockSpec((d, d), lambda i: (0, 0)),
                vec4, vec4, vec4, vecd, vecd,
            ]
        ),
        out_specs=[
            pl.BlockSpec((KF, blk, DS), lambda i: (0, i, 0)),
            pl.BlockSpec((blk, d), lambda i: (i, 0)),
        ],
        out_shape=[
            jax.ShapeDtypeStruct((KF, n, DS), jnp.float32),
            jax.ShapeDtypeStruct((n, d), jnp.float32),
        ],
    )(*accs, y4, dega, degb, w_proj, bg4, s14, t14, s2f, tb2f)


def kernel(x, edge_index, W_gcn, b_gcn, bn1_gamma, bn1_beta, bn1_mean,
           bn1_var, W_proj, b_proj, bn2_gamma, bn2_beta, bn2_mean, bn2_var):
    n, d = x.shape
    e = edge_index.shape[1]
    eps = 1e-5

    cpw = -(-e // (CH * NW))
    cpw = -(-cpw // (2 * NB)) * (2 * NB)
    e_pad = cpw * CH * NW
    n_hist = -(-(n + 16) // (NS * LN)) * (NS * LN)
    pad = e_pad - e

    row = edge_index[0]
    col = edge_index[1]
    if pad:
        row = jnp.concatenate([row, jnp.zeros((pad,), jnp.int32)])
        trash = n + (jnp.arange(pad, dtype=jnp.int32) % jnp.int32(CH))
        col = jnp.concatenate([col, trash])
    row2 = row.reshape(NW * cpw, CH)
    col2 = col.reshape(NW * cpw, CH)

    s1 = bn1_gamma * lax.rsqrt(bn1_var + eps)
    t1 = bn1_beta - bn1_mean * s1
    s2 = bn2_gamma * lax.rsqrt(bn2_var + eps)
    tb2 = (bn2_beta - bn2_mean * s2) + b_proj * s2
    bg4 = b_gcn.reshape(KF, 1, DS)
    s14 = s1.reshape(KF, 1, DS)
    t14 = t1.reshape(KF, 1, DS)
    s2f = s2.reshape(1, d)
    tb2f = tb2.reshape(1, d)

    deg_parts = _degree_kernel(col2, n_hist, cpw)
    dega, degb = deg_parts[0], deg_parts[1]
    y = _pre_kernel(x, W_gcn, dega, degb, 512)
    y4 = y.reshape(n, KF, DS).transpose(1, 0, 2)
    ytabs = [y4[kk] for kk in range(KF)]
    accs = _edge_kernel(ytabs, row2, col2, n_hist, cpw)
    h4, p = _final_kernel(accs, y4, dega, degb, W_proj, bg4, s14, t14,
                          s2f, tb2f, 512)
    h = h4.transpose(1, 0, 2).reshape(n, d)
    return (h, p)


# speedup vs baseline: 1.2796x; 1.2796x over previous
import functools

import jax
import jax.numpy as jnp
from jax import lax
from jax.experimental import pallas as pl
from jax.experimental.pallas import tpu as pltpu
from jax.experimental.pallas import tpu_sc as plsc

NC = 2
NS = 16
LN = 16
NW = NC * NS
CH = 128
KF = 4
DS = 32


def _sc_mesh():
    return plsc.VectorSubcoreMesh(
        core_axis_name="c", subcore_axis_name="s",
        num_cores=NC, num_subcores=NS)


def _degree_kernel(col2, n_hist, cpw):
    rpt = n_hist // NS

    @functools.partial(
        pl.kernel,
        out_type=jax.ShapeDtypeStruct((NC, n_hist), jnp.float32),
        mesh=_sc_mesh(),
        compiler_params=pltpu.CompilerParams(use_tc_tiling_on_sc=False),
        scratch_types=[
            pltpu.VMEM((cpw, CH), jnp.int32),
            pltpu.VMEM((CH,), jnp.float32),
            pltpu.VMEM((rpt,), jnp.float32),
            pltpu.VMEM_SHARED((n_hist,), jnp.float32),
        ],
    )
    def k(col_hbm, deg_hbm, colbuf, ones_v, tbuf, deg_sh):
        ci = lax.axis_index("c")
        s = lax.axis_index("s")
        w = s * NC + ci
        for j in range(CH // LN):
            ones_v[pl.ds(j * LN, LN)] = jnp.ones((LN,), jnp.float32)

        def zero_body(i, _):
            tbuf[pl.ds(i * LN, LN)] = jnp.zeros((LN,), jnp.float32)
            return 0
        lax.fori_loop(0, rpt // LN, zero_body, 0)
        pltpu.sync_copy(tbuf, deg_sh.at[pl.ds(s * rpt, rpt)])
        pltpu.sync_copy(col_hbm.at[pl.ds(w * cpw, cpw)], colbuf)
        plsc.subcore_barrier()

        def body(ch, _):
            pltpu.sync_copy(ones_v, deg_sh.at[colbuf.at[ch]], add=True)
            return 0
        lax.fori_loop(0, cpw, body, 0)
        plsc.subcore_barrier()

        pltpu.sync_copy(deg_sh.at[pl.ds(s * rpt, rpt)], tbuf)
        pltpu.sync_copy(tbuf, deg_hbm.at[ci, pl.ds(s * rpt, rpt)])

    return k(col2)


def _edge_kernel(ytabs, row2, col2, n_hist, cpw):
    rpt = n_hist // NS

    @functools.partial(
        pl.kernel,
        out_type=[jax.ShapeDtypeStruct((NC, n_hist, DS), jnp.float32)
                  for _ in range(KF)],
        mesh=_sc_mesh(),
        compiler_params=pltpu.CompilerParams(use_tc_tiling_on_sc=False),
        scratch_types=[
            pltpu.VMEM((cpw, CH), jnp.int32),
            pltpu.VMEM((cpw, CH), jnp.int32),
            pltpu.VMEM((CH, DS), jnp.float32),
            pltpu.VMEM((rpt, DS), jnp.float32),
            pltpu.VMEM_SHARED((n_hist, DS), jnp.float32),
            pltpu.SemaphoreType.DMA,
        ],
    )
    def k(y0, y1, y2, y3, row_hbm, col_hbm, a0, a1, a2, a3,
          rowbuf, colbuf, rows_v, zbuf, acc_sh, sem):
        ci = lax.axis_index("c")
        s = lax.axis_index("s")
        w = s * NC + ci
        ys = [y0, y1, y2, y3]
        accs = [a0, a1, a2, a3]

        pltpu.sync_copy(row_hbm.at[pl.ds(w * cpw, cpw)], rowbuf)
        pltpu.sync_copy(col_hbm.at[pl.ds(w * cpw, cpw)], colbuf)

        def zero_body(i, _):
            for j in range(DS // LN):
                zbuf[i, pl.ds(j * LN, LN)] = jnp.zeros((LN,), jnp.float32)
            return 0
        lax.fori_loop(0, rpt, zero_body, 0)

        for kk in range(KF):
            pltpu.sync_copy(zbuf, acc_sh.at[pl.ds(s * rpt, rpt)])
            plsc.subcore_barrier()
            y_t = ys[kk]

            def body(ch, _, _y=y_t):
                pltpu.async_copy(_y.at[rowbuf.at[ch]], rows_v, sem).wait()
                pltpu.sync_copy(rows_v, acc_sh.at[colbuf.at[ch]], add=True)
                return 0
            lax.fori_loop(0, cpw, body, 0)
            plsc.subcore_barrier()

            pltpu.sync_copy(acc_sh.at[pl.ds(s * rpt, rpt)], zbuf)
            pltpu.sync_copy(zbuf, accs[kk].at[ci, pl.ds(s * rpt, rpt)])
            if kk + 1 < KF:
                lax.fori_loop(0, rpt, zero_body, 0)

    return k(*ytabs, row2, col2)


def _pre_kernel(x, w_gcn, dega, degb, blk):
    n, d = x.shape
    n_hist = dega.shape[0]
    grid = (n + blk - 1) // blk

    def body(x_ref, w_ref, da_ref, db_ref, y_ref):
        i = pl.program_id(0)
        deg = da_ref[pl.ds(i * blk, blk)] + db_ref[pl.ds(i * blk, blk)] + 1.0
        dinv = lax.rsqrt(deg)
        xw = jnp.dot(x_ref[...], w_ref[...],
                     preferred_element_type=jnp.float32)
        y_ref[...] = xw * dinv[:, None]

    return pl.pallas_call(
        body,
        grid=(grid,),
        in_specs=[
            pl.BlockSpec((blk, d), lambda i: (i, 0)),
            pl.BlockSpec((d, d), lambda i: (0, 0)),
            pl.BlockSpec((n_hist,), lambda i: (0,)),
            pl.BlockSpec((n_hist,), lambda i: (0,)),
        ],
        out_specs=pl.BlockSpec((blk, d), lambda i: (i, 0)),
        out_shape=jax.ShapeDtypeStruct((n, d), jnp.float32),
    )(x, w_gcn, dega, degb)


def _final_kernel(accs, y4, dega, degb, w_proj, bg4, s14, t14, s2f, tb2f,
                  blk):
    n = y4.shape[1]
    d = KF * DS
    n_hist = dega.shape[0]
    grid = (n + blk - 1) // blk

    def body(a0, a1, a2, a3, y_ref, da_ref, db_ref, w_ref,
             bg_ref, s1_ref, t1_ref, s2_ref, tb2_ref, h_ref, p_ref):
        i = pl.program_id(0)
        deg = da_ref[pl.ds(i * blk, blk)] + db_ref[pl.ds(i * blk, blk)] + 1.0
        dinv = lax.rsqrt(deg)[:, None]
        acc_refs = [a0, a1, a2, a3]
        z = jnp.zeros((blk, d), jnp.float32)
        for kk in range(KF):
            acc = acc_refs[kk][0] + acc_refs[kk][1] + y_ref[kk]
            b_gcn = bg_ref[kk, 0, :][None, :]
            s1 = s1_ref[kk, 0, :][None, :]
            t1 = t1_ref[kk, 0, :][None, :]
            h_k = (acc * dinv + b_gcn) * s1 + t1
            h_ref[kk] = h_k
            z = z + jnp.dot(h_k, w_ref[pl.ds(kk * DS, DS), :],
                            preferred_element_type=jnp.float32)
        p_ref[...] = jnp.maximum(z * s2_ref[0, :][None, :]
                                 + tb2_ref[0, :][None, :], 0.0)

    vec4 = pl.BlockSpec((KF, 1, DS), lambda i: (0, 0, 0))
    vecd = pl.BlockSpec((1, d), lambda i: (0, 0))
    return pl.pallas_call(
        body,
        grid=(grid,),
        in_specs=(
            [pl.BlockSpec((NC, blk, DS), lambda i: (0, i, 0))
             for _ in range(KF)]
            + [
                pl.BlockSpec((KF, blk, DS), lambda i: (0, i, 0)),
                pl.BlockSpec((n_hist,), lambda i: (0,)),
                pl.BlockSpec((n_hist,), lambda i: (0,)),
                pl.BlockSpec((d, d), lambda i: (0, 0)),
                vec4, vec4, vec4, vecd, vecd,
            ]
        ),
        out_specs=[
            pl.BlockSpec((KF, blk, DS), lambda i: (0, i, 0)),
            pl.BlockSpec((blk, d), lambda i: (i, 0)),
        ],
        out_shape=[
            jax.ShapeDtypeStruct((KF, n, DS), jnp.float32),
            jax.ShapeDtypeStruct((n, d), jnp.float32),
        ],
    )(*accs, y4, dega, degb, w_proj, bg4, s14, t14, s2f, tb2f)


def kernel(x, edge_index, W_gcn, b_gcn, bn1_gamma, bn1_beta, bn1_mean,
           bn1_var, W_proj, b_proj, bn2_gamma, bn2_beta, bn2_mean, bn2_var):
    n, d = x.shape
    e = edge_index.shape[1]
    eps = 1e-5

    cpw = -(-e // (CH * NW))
    e_pad = cpw * CH * NW
    n_hist = -(-(n + 16) // (NS * LN)) * (NS * LN)
    pad = e_pad - e

    row = edge_index[0]
    col = edge_index[1]
    if pad:
        row = jnp.concatenate([row, jnp.zeros((pad,), jnp.int32)])
        trash = n + (jnp.arange(pad, dtype=jnp.int32) % jnp.int32(CH))
        col = jnp.concatenate([col, trash])
    row2 = row.reshape(NW * cpw, CH)
    col2 = col.reshape(NW * cpw, CH)

    s1 = bn1_gamma * lax.rsqrt(bn1_var + eps)
    t1 = bn1_beta - bn1_mean * s1
    s2 = bn2_gamma * lax.rsqrt(bn2_var + eps)
    tb2 = (bn2_beta - bn2_mean * s2) + b_proj * s2
    bg4 = b_gcn.reshape(KF, 1, DS)
    s14 = s1.reshape(KF, 1, DS)
    t14 = t1.reshape(KF, 1, DS)
    s2f = s2.reshape(1, d)
    tb2f = tb2.reshape(1, d)

    deg_parts = _degree_kernel(col2, n_hist, cpw)
    dega, degb = deg_parts[0], deg_parts[1]
    y = _pre_kernel(x, W_gcn, dega, degb, 512)
    y4 = y.reshape(n, KF, DS).transpose(1, 0, 2)
    ytabs = [y4[kk] for kk in range(KF)]
    accs = _edge_kernel(ytabs, row2, col2, n_hist, cpw)
    h4, p = _final_kernel(accs, y4, dega, degb, W_proj, bg4, s14, t14,
                          s2f, tb2f, 512)
    h = h4.transpose(1, 0, 2).reshape(n, d)
    return (h, p)
